# baseline (speedup 1.0000x reference)
import functools

import jax
import jax.numpy as jnp
from jax import lax
from jax.experimental import pallas as pl
from jax.experimental.pallas import tpu as pltpu
from jax.experimental.pallas import tpu_sc as plsc

B = 16384
D = 64
K = 20
NW = 32
NB = B // NW
C = 32
NCH = NB // C
CLAMP = 4.0

_LOG1P = (
    2.38672349e-09, 9.99999671e-01, -4.99988787e-01, 3.33167154e-01,
    -2.48659331e-01, 1.93379621e-01, -1.45182331e-01, 9.47104384e-02,
    -4.71380052e-02, 1.51470705e-02, -2.28832363e-03,
)


def _softplus(v):
    u = jnp.exp(-jnp.abs(v))
    acc = jnp.full((16,), _LOG1P[-1], jnp.float32)
    for coef in _LOG1P[-2::-1]:
        acc = acc * u + jnp.float32(coef)
    return jnp.maximum(v, 0.0) + acc


def _sc_body(in_idx_hbm, pos_idx_hbm, neg_idx_hbm, emb_hbm, oemb_hbm, out_hbm,
             in_idx_v, pos_idx_v, neg_idx_v, in_rows_v, pos_rows_v,
             neg_rows_v, out_v, sem):
    wid = lax.axis_index("s") * 2 + lax.axis_index("c")
    r0 = pl.multiple_of(wid * (NB // 128), NB // 128)
    pltpu.sync_copy(in_idx_hbm.at[pl.ds(r0, NB // 128)], in_idx_v)
    pltpu.sync_copy(pos_idx_hbm.at[pl.ds(r0, NB // 128)], pos_idx_v)
    nr0 = pl.multiple_of(wid * (NB * K // 128), NB * K // 128)
    pltpu.sync_copy(neg_idx_hbm.at[pl.ds(nr0, NB * K // 128)], neg_idx_v)
    for j in range(NB // 128):
        pltpu.async_copy(emb_hbm.at[in_idx_v.at[j]],
                         in_rows_v.at[pl.ds(j * 128, 128)], sem).wait()
        pltpu.async_copy(oemb_hbm.at[pos_idx_v.at[j]],
                         pos_rows_v.at[pl.ds(j * 128, 128)], sem).wait()

    lane = lax.iota(jnp.int32, 16)
    himask = jnp.where(lane < K - 15, 1.0, 0.0).astype(jnp.float32)
    zeros = jnp.zeros((16,), jnp.float32)

    def chunk_body(c, carry):
        cr = pl.multiple_of(c * (C * K // 128), C * K // 128)
        for j in range(C * K // 128):
            pltpu.async_copy(oemb_hbm.at[neg_idx_v.at[cr + j]],
                             neg_rows_v.at[pl.ds(j * 128, 128)], sem).wait()

        def b_body(bl, res):
            gb = c * C + bl
            i0 = in_rows_v[gb, pl.ds(0, 16)]
            i1 = in_rows_v[gb, pl.ds(16, 16)]
            i2 = in_rows_v[gb, pl.ds(32, 16)]
            i3 = in_rows_v[gb, pl.ds(48, 16)]
            p0 = pos_rows_v[gb, pl.ds(0, 16)]
            p1 = pos_rows_v[gb, pl.ds(16, 16)]
            p2 = pos_rows_v[gb, pl.ds(32, 16)]
            p3 = pos_rows_v[gb, pl.ds(48, 16)]
            s = jnp.sum(i0 * p0 + i1 * p1 + i2 * p2 + i3 * p3)
            v0 = jnp.where(lane == 0, -jnp.clip(s, -CLAMP, CLAMP), zeros)
            v1 = zeros
            for k in range(K):
                rb = bl * K + k
                n0 = neg_rows_v[rb, pl.ds(0, 16)]
                n1 = neg_rows_v[rb, pl.ds(16, 16)]
                n2 = neg_rows_v[rb, pl.ds(32, 16)]
                n3 = neg_rows_v[rb, pl.ds(48, 16)]
                s = jnp.clip(jnp.sum(i0 * n0 + i1 * n1 + i2 * n2 + i3 * n3),
                             -CLAMP, CLAMP)
                if k < 15:
                    v0 = jnp.where(lane == 1 + k, s, v0)
                else:
                    v1 = jnp.where(lane == k - 15, s, v1)
            total = jnp.sum(_softplus(v0) + _softplus(v1) * himask)
            res = jnp.where(lane == lax.rem(bl, 16), total, res)

            @pl.when(lax.rem(bl, 16) == 15)
            def _store():
                ob = pl.multiple_of(c * C + bl - 15, 16)
                out_v[pl.ds(ob, 16)] = res

            return res

        lax.fori_loop(0, C, b_body, zeros)
        return carry

    lax.fori_loop(0, NCH, chunk_body, 0)
    ob = pl.multiple_of(wid * NB, NB)
    pltpu.sync_copy(out_v, out_hbm.at[pl.ds(ob, NB)])


_sc_kernel = functools.partial(
    pl.kernel,
    mesh=plsc.VectorSubcoreMesh(core_axis_name="c", subcore_axis_name="s"),
    out_type=jax.ShapeDtypeStruct((B,), jnp.float32),
    compiler_params=pltpu.CompilerParams(
        needs_layout_passes=False, use_tc_tiling_on_sc=False),
    scratch_types=[
        pltpu.VMEM((NB // 128, 128), jnp.int32),
        pltpu.VMEM((NB // 128, 128), jnp.int32),
        pltpu.VMEM((NB * K // 128, 128), jnp.int32),
        pltpu.VMEM((NB, D), jnp.float32),
        pltpu.VMEM((NB, D), jnp.float32),
        pltpu.VMEM((C * K, D), jnp.float32),
        pltpu.VMEM((NB,), jnp.float32),
        pltpu.SemaphoreType.DMA,
    ],
)(_sc_body)


@jax.jit
def kernel(inputs, positiveOutputs, negativeOutputs, emb_weight,
           out_emb_weight):
    in_idx = inputs.astype(jnp.int32).reshape(B // 128, 128)
    pos_idx = positiveOutputs.astype(jnp.int32).reshape(B // 128, 128)
    neg_idx = negativeOutputs.astype(jnp.int32).reshape(B * K // 128, 128)
    return _sc_kernel(in_idx, pos_idx, neg_idx, emb_weight, out_emb_weight)
